# Initial kernel scaffold; baseline (speedup 1.0000x reference)
import jax
import jax.numpy as jnp
from jax.experimental import pallas as pl


def kernel(input_ids, semantic_table, pos_table):
    raise NotImplementedError("write your pallas kernel here")


# speedup vs baseline: 2.0087x; 2.0087x over previous
import functools

import jax
import jax.numpy as jnp
from jax import lax
from jax.experimental import pallas as pl
from jax.experimental.pallas import tpu as pltpu
from jax.experimental.pallas import tpu_sc as plsc

NC = 2
NS = 16
LANES = 16


def _make_kernel(N, V, D, P, T, n_per_w, C, G):
    n_chunks = n_per_w // C
    mesh = plsc.VectorSubcoreMesh(
        core_axis_name="c", subcore_axis_name="s", num_cores=NC, num_subcores=NS
    )

    @functools.partial(
        pl.kernel,
        mesh=mesh,
        compiler_params=pltpu.CompilerParams(use_tc_tiling_on_sc=False),
        out_type=jax.ShapeDtypeStruct((N, D), jnp.float32),
        scratch_types=[
            pltpu.VMEM((T, D), jnp.float32),
            pltpu.VMEM((C,), jnp.int32),
            pltpu.VMEM((C, D), jnp.float32),
            pltpu.SemaphoreType.DMA,
        ],
    )
    def ker(ids_hbm, tab_hbm, pos_hbm, out_hbm, pos_v, idx_v, rows_v, sem):
        wid = lax.axis_index("s") * NC + lax.axis_index("c")
        base = wid * n_per_w
        pltpu.sync_copy(pos_hbm.at[pl.ds(0, T)], pos_v)

        def chunk_body(g, carry):
            off = base + g * C
            pltpu.sync_copy(ids_hbm.at[pl.ds(off, C)], idx_v)
            copies = [
                pltpu.async_copy(
                    tab_hbm.at[idx_v.at[pl.ds(j * G, G)]],
                    rows_v.at[pl.ds(j * G, G)],
                    sem,
                )
                for j in range(C // G)
            ]
            for cp in copies:
                cp.wait()

            t0 = lax.rem(g * C, T)

            def row_body(r, carry2):
                t = lax.rem(t0 + r, T)
                for j in range(D // LANES):
                    sl = pl.ds(j * LANES, LANES)
                    rows_v[r, sl] = rows_v[r, sl] + pos_v[t, sl]
                return carry2

            lax.fori_loop(0, C, row_body, 0)
            pltpu.sync_copy(rows_v, out_hbm.at[pl.ds(off, C)])
            return carry

        lax.fori_loop(0, n_chunks, chunk_body, 0)

    return ker


def kernel(input_ids, semantic_table, pos_table):
    B, T = input_ids.shape
    V, D = semantic_table.shape
    P = pos_table.shape[0]
    N = B * T
    NW = NC * NS
    n_per_w = N // NW
    C = 512
    G = 128
    assert N % NW == 0 and n_per_w % C == 0 and C % G == 0 and D % LANES == 0

    ker = _make_kernel(N, V, D, P, T, n_per_w, C, G)
    out_flat = ker(input_ids.reshape(N), semantic_table, pos_table)
    return out_flat.reshape(B, T, D)


# speedup vs baseline: 2.1039x; 1.0474x over previous
import functools

import jax
import jax.numpy as jnp
from jax import lax
from jax.experimental import pallas as pl
from jax.experimental.pallas import tpu as pltpu
from jax.experimental.pallas import tpu_sc as plsc

NC = 2
NS = 16
LANES = 16


def _make_kernel(N, V, D, P, T, n_per_w, C, G):
    n_chunks = n_per_w // C
    mesh = plsc.VectorSubcoreMesh(
        core_axis_name="c", subcore_axis_name="s", num_cores=NC, num_subcores=NS
    )

    @functools.partial(
        pl.kernel,
        mesh=mesh,
        compiler_params=pltpu.CompilerParams(use_tc_tiling_on_sc=False),
        out_type=jax.ShapeDtypeStruct((N, D), jnp.float32),
        scratch_types=[
            pltpu.VMEM((T, D), jnp.float32),
            pltpu.VMEM((n_per_w,), jnp.int32),
            pltpu.VMEM((C, D), jnp.float32),
            pltpu.VMEM((C, D), jnp.float32),
            pltpu.SemaphoreType.DMA,
            pltpu.SemaphoreType.DMA,
            pltpu.SemaphoreType.DMA,
            pltpu.SemaphoreType.DMA,
        ],
    )
    def ker(ids_hbm, tab_hbm, pos_hbm, out_hbm, pos_v, idx_v,
            rows0, rows1, gsem0, gsem1, osem0, osem1):
        rows = (rows0, rows1)
        gsem = (gsem0, gsem1)
        osem = (osem0, osem1)
        wid = lax.axis_index("s") * NC + lax.axis_index("c")
        base = wid * n_per_w
        pltpu.sync_copy(pos_hbm.at[pl.ds(0, T)], pos_v)
        pltpu.sync_copy(ids_hbm.at[pl.ds(base, n_per_w)], idx_v)

        def fire_gathers(g, b):
            for j in range(C // G):
                pltpu.async_copy(
                    tab_hbm.at[idx_v.at[pl.ds(g * C + j * G, G)]],
                    rows[b].at[pl.ds(j * G, G)],
                    gsem[b],
                )

        def drain(sem, b):
            pltpu.make_async_copy(tab_hbm.at[pl.ds(0, C)], rows[b], sem).wait()

        fire_gathers(0, 0)

        def pair_body(gp, carry):
            for b in (0, 1):
                g = gp * 2 + b
                o = 1 - b
                drain(gsem[b], b)
                t0 = lax.rem(g * C, T)

                def row_body(r, c2):
                    t = lax.rem(t0 + r, T)
                    for j in range(D // LANES):
                        sl = pl.ds(j * LANES, LANES)
                        rows[b][r, sl] = rows[b][r, sl] + pos_v[t, sl]
                    return c2

                lax.fori_loop(0, C, row_body, 0)

                @pl.when(jnp.logical_and(g >= 1, g + 1 < n_chunks))
                def _():
                    drain(osem[o], o)

                @pl.when(g + 1 < n_chunks)
                def _():
                    fire_gathers(g + 1, o)

                pltpu.async_copy(
                    rows[b], out_hbm.at[pl.ds(base + g * C, C)], osem[b]
                )
            return carry

        lax.fori_loop(0, n_chunks // 2, pair_body, 0)
        drain(osem[0], 0)
        drain(osem[1], 1)

    return ker


def kernel(input_ids, semantic_table, pos_table):
    B, T = input_ids.shape
    V, D = semantic_table.shape
    P = pos_table.shape[0]
    N = B * T
    NW = NC * NS
    n_per_w = N // NW
    C = 512
    G = 128
    assert N % NW == 0 and n_per_w % C == 0 and C % G == 0 and D % LANES == 0
    assert (n_per_w // C) % 2 == 0

    ker = _make_kernel(N, V, D, P, T, n_per_w, C, G)
    out_flat = ker(input_ids.reshape(N), semantic_table, pos_table)
    return out_flat.reshape(B, T, D)


# speedup vs baseline: 2.6398x; 1.2547x over previous
import functools

import jax
import jax.numpy as jnp
from jax import lax
from jax.experimental import pallas as pl
from jax.experimental.pallas import tpu as pltpu
from jax.experimental.pallas import tpu_sc as plsc

NC = 2
NS = 16
LANES = 16


def _make_kernel(N, V, D, P, T, n_per_w, C, G):
    n_chunks = n_per_w // C
    mesh = plsc.VectorSubcoreMesh(
        core_axis_name="c", subcore_axis_name="s", num_cores=NC, num_subcores=NS
    )

    @functools.partial(
        pl.kernel,
        mesh=mesh,
        compiler_params=pltpu.CompilerParams(use_tc_tiling_on_sc=False),
        out_type=jax.ShapeDtypeStruct((N, D), jnp.float32),
        scratch_types=[
            pltpu.VMEM((C, D), jnp.float32),
            pltpu.VMEM((n_per_w,), jnp.int32),
            pltpu.VMEM((C, D), jnp.float32),
            pltpu.VMEM((C, D), jnp.float32),
            pltpu.SemaphoreType.DMA,
            pltpu.SemaphoreType.DMA,
            pltpu.SemaphoreType.DMA,
            pltpu.SemaphoreType.DMA,
        ],
    )
    def ker(ids_hbm, tab_hbm, pos_hbm, out_hbm, pos_v, idx_v,
            rows0, rows1, gsem0, gsem1, osem0, osem1):
        rows = (rows0, rows1)
        gsem = (gsem0, gsem1)
        osem = (osem0, osem1)
        wid = lax.axis_index("s") * NC + lax.axis_index("c")
        base = wid * n_per_w
        for k in range(C // T):
            pltpu.sync_copy(pos_hbm.at[pl.ds(0, T)], pos_v.at[pl.ds(k * T, T)])
        pltpu.sync_copy(ids_hbm.at[pl.ds(base, n_per_w)], idx_v)

        def fire_gathers(g, b):
            for j in range(C // G):
                pltpu.async_copy(
                    tab_hbm.at[idx_v.at[pl.ds(g * C + j * G, G)]],
                    rows[b].at[pl.ds(j * G, G)],
                    gsem[b],
                )

        def drain(sem, b):
            pltpu.make_async_copy(tab_hbm.at[pl.ds(0, C)], rows[b], sem).wait()

        fire_gathers(0, 0)

        def pair_body(gp, carry):
            for b in (0, 1):
                g = gp * 2 + b
                o = 1 - b
                drain(gsem[b], b)

                @plsc.parallel_loop(0, C, 1, unroll=8)
                def row_body(r):
                    for j in range(D // LANES):
                        sl = pl.ds(j * LANES, LANES)
                        rows[b][r, sl] = rows[b][r, sl] + pos_v[r, sl]

                @pl.when(jnp.logical_and(g >= 1, g + 1 < n_chunks))
                def _():
                    drain(osem[o], o)

                @pl.when(g + 1 < n_chunks)
                def _():
                    fire_gathers(g + 1, o)

                pltpu.async_copy(
                    rows[b], out_hbm.at[pl.ds(base + g * C, C)], osem[b]
                )
            return carry

        lax.fori_loop(0, n_chunks // 2, pair_body, 0)
        drain(osem[0], 0)
        drain(osem[1], 1)

    return ker


def kernel(input_ids, semantic_table, pos_table):
    B, T = input_ids.shape
    V, D = semantic_table.shape
    P = pos_table.shape[0]
    N = B * T
    NW = NC * NS
    n_per_w = N // NW
    C = 2 * T
    G = 80
    assert N % NW == 0 and n_per_w % C == 0 and C % G == 0 and D % LANES == 0
    assert (n_per_w // C) % 2 == 0 and G % 8 == 0

    ker = _make_kernel(N, V, D, P, T, n_per_w, C, G)
    out_flat = ker(input_ids.reshape(N), semantic_table, pos_table)
    return out_flat.reshape(B, T, D)
